# baseline (speedup 1.0000x reference)
import functools

import jax
import jax.numpy as jnp
from jax import lax
from jax.experimental import pallas as pl
from jax.experimental.pallas import tpu as pltpu
from jax.experimental.pallas import tpu_sc as plsc

B = 16384
D_OUT = 151
NC, NS, NL = 2, 16, 16
NW = NC * NS
B_PER_W = B // NW
BATCH = 128
N_BATCH = B_PER_W // BATCH
ROW_OFF = (0, 128, 384)


def _body(cat_hbm, wb_hbm, ws_hbm, wt_hbm, wd_hbm, out_hbm,
          idxa_v, idxb_v, dir_v, tab_v, acc0_v, acc1_v,
          sema, semb, sem0, sem1):
    wid = lax.axis_index("s") * NC + lax.axis_index("c")
    base = wid * B_PER_W

    idxs = (idxa_v, idxb_v)
    isems = (sema, semb)
    icps = [None, None]
    icps[0] = pltpu.async_copy(cat_hbm.at[pl.ds(base, BATCH)], idxa_v, sema)
    pltpu.sync_copy(wd_hbm, dir_v)
    pltpu.sync_copy(wb_hbm, tab_v.at[pl.ds(0, 128 * 50)])
    pltpu.sync_copy(ws_hbm, tab_v.at[pl.ds(128 * 50, 256 * 50)])
    pltpu.sync_copy(wt_hbm, tab_v.at[pl.ds(384 * 50, 128 * 50)])

    iota = lax.iota(jnp.int32, NL)
    zero = jnp.zeros((NL,), jnp.int32)
    kcol = [zero, zero + 1, zero + 2, zero + 3]
    cols = [iota, iota + 16, iota + 32]
    dcols = [[iota + (k * 50 + c * 16) for c in range(3)] for k in range(3)]

    def repack_batch(idx_v, acc_v):
        @plsc.parallel_loop(0, BATCH, unroll=4)
        def _rows(b):
            bv = jnp.full((NL,), b, jnp.int32)
            for k in range(3):
                tk = plsc.load_gather(idx_v, [bv, kcol[k]])
                tq = tk * 50 + (ROW_OFF[k] * 50)
                for c in range(3):
                    v = plsc.load_gather(tab_v, [tq + cols[c]])
                    plsc.store_scatter(acc_v, [bv, dcols[k][c]], v)

        @plsc.parallel_loop(0, BATCH // NL, unroll=2)
        def _tails(j):
            rl = j * NL + iota
            for k in range(3):
                tkv = plsc.load_gather(idx_v, [rl, kcol[k]])
                tkq = tkv * 50 + (ROW_OFF[k] * 50)
                for c in (48, 49):
                    v = plsc.load_gather(tab_v, [tkq + c])
                    plsc.store_scatter(
                        acc_v, [rl, jnp.full((NL,), k * 50 + c, jnp.int32)], v)
            dv = plsc.load_gather(idx_v, [rl, kcol[3]])
            dvals = plsc.load_gather(dir_v, [dv, zero])
            plsc.store_scatter(
                acc_v, [rl, jnp.full((NL,), 150, jnp.int32)], dvals)

    accs = (acc0_v, acc1_v)
    osems = (sem0, sem1)
    ocps = [None, None]
    for t in range(N_BATCH):
        if t + 1 < N_BATCH:
            icps[(t + 1) % 2] = pltpu.async_copy(
                cat_hbm.at[pl.ds(base + (t + 1) * BATCH, BATCH)],
                idxs[(t + 1) % 2], isems[(t + 1) % 2])
        if ocps[t % 2] is not None:
            ocps[t % 2].wait()
        icps[t % 2].wait()
        repack_batch(idxs[t % 2], accs[t % 2])
        ocps[t % 2] = pltpu.async_copy(
            accs[t % 2], out_hbm.at[pl.ds(base + t * BATCH, BATCH)],
            osems[t % 2])
    ocps[0].wait()
    ocps[1].wait()


@jax.jit
def _run(cat_tensor, wb_flat, ws_flat, wt_flat, wd):
    mesh = plsc.VectorSubcoreMesh(core_axis_name="c", subcore_axis_name="s")
    return pl.kernel(
        _body,
        out_type=jax.ShapeDtypeStruct((B, D_OUT), jnp.float32),
        mesh=mesh,
        scratch_types=[
            pltpu.VMEM((BATCH, 4), jnp.int32),
            pltpu.VMEM((BATCH, 4), jnp.int32),
            pltpu.VMEM((2, 1), jnp.float32),
            pltpu.VMEM((512 * 50,), jnp.float32),
            pltpu.VMEM((BATCH, D_OUT), jnp.float32),
            pltpu.VMEM((BATCH, D_OUT), jnp.float32),
            pltpu.SemaphoreType.DMA,
            pltpu.SemaphoreType.DMA,
            pltpu.SemaphoreType.DMA,
            pltpu.SemaphoreType.DMA,
        ],
        compiler_params=pltpu.CompilerParams(
            use_tc_tiling_on_sc=False, needs_layout_passes=False),
    )(cat_tensor, wb_flat, ws_flat, wt_flat, wd)


def kernel(cat_tensor, W_bus_id, W_station_id, W_time_period, W_direction):
    return _run(cat_tensor, W_bus_id.reshape(-1), W_station_id.reshape(-1),
                W_time_period.reshape(-1), W_direction)


# speedup vs baseline: 1.5566x; 1.5566x over previous
import functools

import jax
import jax.numpy as jnp
from jax import lax
from jax.experimental import pallas as pl
from jax.experimental.pallas import tpu as pltpu
from jax.experimental.pallas import tpu_sc as plsc

B = 16384
D_OUT = 151
NC, NS, NL = 2, 16, 16
NW = NC * NS
B_PER_W = B // NW
BATCH = 128
N_BATCH = B_PER_W // BATCH
ROW_OFF = (0, 128, 384)


def _body(cat_hbm, wb_hbm, ws_hbm, wt_hbm, wd_hbm, out_hbm,
          idxa_v, idxb_v, dir_v, tab_v, acc0_v, acc1_v,
          sema, semb, sem0, sem1):
    wid = lax.axis_index("s") * NC + lax.axis_index("c")
    base = wid * B_PER_W

    idxs = (idxa_v, idxb_v)
    isems = (sema, semb)
    icps = [None, None]
    icps[0] = pltpu.async_copy(cat_hbm.at[pl.ds(base, BATCH)], idxa_v, sema)
    pltpu.sync_copy(wd_hbm, dir_v)
    pltpu.sync_copy(wb_hbm, tab_v.at[pl.ds(0, 128 * 50)])
    pltpu.sync_copy(ws_hbm, tab_v.at[pl.ds(128 * 50, 256 * 50)])
    pltpu.sync_copy(wt_hbm, tab_v.at[pl.ds(384 * 50, 128 * 50)])

    iota = lax.iota(jnp.int32, NL)
    zero = jnp.zeros((NL,), jnp.int32)
    kcol = [zero, zero + 1, zero + 2, zero + 3]
    cols = [iota, iota + 16, iota + 32]
    dcols = [[iota + (k * 50 + c * 16) for c in range(3)] for k in range(3)]

    def repack_batch(idx_v, acc_v):
        @plsc.parallel_loop(0, BATCH, unroll=2)
        def _rows(b):
            bv = jnp.full((NL,), b, jnp.int32)
            for k in range(3):
                tk = plsc.load_gather(idx_v, [bv, kcol[k]])
                tq = tk * 50 + (ROW_OFF[k] * 50)
                for c in range(3):
                    v = plsc.load_gather(tab_v, [tq + cols[c]])
                    plsc.store_scatter(acc_v, [bv, dcols[k][c]], v)

        @plsc.parallel_loop(0, BATCH // NL, unroll=2)
        def _tails(j):
            rl = j * NL + iota
            for k in range(3):
                tkv = plsc.load_gather(idx_v, [rl, kcol[k]])
                tkq = tkv * 50 + (ROW_OFF[k] * 50)
                for c in (48, 49):
                    v = plsc.load_gather(tab_v, [tkq + c])
                    plsc.store_scatter(
                        acc_v, [rl, jnp.full((NL,), k * 50 + c, jnp.int32)], v)
            dv = plsc.load_gather(idx_v, [rl, kcol[3]])
            dvals = plsc.load_gather(dir_v, [dv, zero])
            plsc.store_scatter(
                acc_v, [rl, jnp.full((NL,), 150, jnp.int32)], dvals)

    accs = (acc0_v, acc1_v)
    osems = (sem0, sem1)
    ocps = [None, None]
    for t in range(N_BATCH):
        if t + 1 < N_BATCH:
            icps[(t + 1) % 2] = pltpu.async_copy(
                cat_hbm.at[pl.ds(base + (t + 1) * BATCH, BATCH)],
                idxs[(t + 1) % 2], isems[(t + 1) % 2])
        if ocps[t % 2] is not None:
            ocps[t % 2].wait()
        icps[t % 2].wait()
        repack_batch(idxs[t % 2], accs[t % 2])
        ocps[t % 2] = pltpu.async_copy(
            accs[t % 2], out_hbm.at[pl.ds(base + t * BATCH, BATCH)],
            osems[t % 2])
    ocps[0].wait()
    ocps[1].wait()


@jax.jit
def _run(cat_tensor, wb_flat, ws_flat, wt_flat, wd):
    mesh = plsc.VectorSubcoreMesh(core_axis_name="c", subcore_axis_name="s")
    return pl.kernel(
        _body,
        out_type=jax.ShapeDtypeStruct((B, D_OUT), jnp.float32),
        mesh=mesh,
        scratch_types=[
            pltpu.VMEM((BATCH, 4), jnp.int32),
            pltpu.VMEM((BATCH, 4), jnp.int32),
            pltpu.VMEM((2, 1), jnp.float32),
            pltpu.VMEM((512 * 50,), jnp.float32),
            pltpu.VMEM((BATCH, D_OUT), jnp.float32),
            pltpu.VMEM((BATCH, D_OUT), jnp.float32),
            pltpu.SemaphoreType.DMA,
            pltpu.SemaphoreType.DMA,
            pltpu.SemaphoreType.DMA,
            pltpu.SemaphoreType.DMA,
        ],
        compiler_params=pltpu.CompilerParams(
            use_tc_tiling_on_sc=True, needs_layout_passes=False),
    )(cat_tensor, wb_flat, ws_flat, wt_flat, wd)


def kernel(cat_tensor, W_bus_id, W_station_id, W_time_period, W_direction):
    return _run(cat_tensor, W_bus_id.reshape(-1), W_station_id.reshape(-1),
                W_time_period.reshape(-1), W_direction)


# speedup vs baseline: 1.6056x; 1.0314x over previous
import functools

import jax
import jax.numpy as jnp
from jax import lax
from jax.experimental import pallas as pl
from jax.experimental.pallas import tpu as pltpu
from jax.experimental.pallas import tpu_sc as plsc

B = 16384
D_OUT = 151
NC, NS, NL = 2, 16, 16
NW = NC * NS
B_PER_W = B // NW
BATCH = 128
N_BATCH = B_PER_W // BATCH
ROW_OFF = (0, 128, 384)


def _body(cat_hbm, tab_hbm, wd_hbm, out_hbm,
          idxa_v, idxb_v, dir_v, tab_v, acc0_v, acc1_v,
          sema, semb, sem0, sem1):
    wid = lax.axis_index("s") * NC + lax.axis_index("c")
    base = wid * B_PER_W

    idxs = (idxa_v, idxb_v)
    isems = (sema, semb)
    icps = [None, None]
    icps[0] = pltpu.async_copy(cat_hbm.at[pl.ds(base, BATCH)], idxa_v, sema)
    pltpu.sync_copy(wd_hbm, dir_v)
    pltpu.sync_copy(tab_hbm, tab_v)

    iota = lax.iota(jnp.int32, NL)
    zero = jnp.zeros((NL,), jnp.int32)
    kcol = [zero, zero + 1, zero + 2, zero + 3]
    cols = [iota, iota + 16, iota + 32]
    dcols = [[iota + (k * 50 + c * 16) for c in range(3)] for k in range(3)]

    def repack_batch(idx_v, acc_v):
        @plsc.parallel_loop(0, BATCH, unroll=2)
        def _rows(b):
            bv = jnp.full((NL,), b, jnp.int32)
            for k in range(3):
                tk = plsc.load_gather(idx_v, [bv, kcol[k]])
                tq = tk * 50 + (ROW_OFF[k] * 50)
                for c in range(3):
                    v = plsc.load_gather(tab_v, [tq + cols[c]])
                    plsc.store_scatter(acc_v, [bv, dcols[k][c]], v)

        @plsc.parallel_loop(0, BATCH // NL, unroll=2)
        def _tails(j):
            rl = j * NL + iota
            for k in range(3):
                tkv = plsc.load_gather(idx_v, [rl, kcol[k]])
                tkq = tkv * 50 + (ROW_OFF[k] * 50)
                for c in (48, 49):
                    v = plsc.load_gather(tab_v, [tkq + c])
                    plsc.store_scatter(
                        acc_v, [rl, jnp.full((NL,), k * 50 + c, jnp.int32)], v)
            dv = plsc.load_gather(idx_v, [rl, kcol[3]])
            dvals = plsc.load_gather(dir_v, [dv, zero])
            plsc.store_scatter(
                acc_v, [rl, jnp.full((NL,), 150, jnp.int32)], dvals)

    accs = (acc0_v, acc1_v)
    osems = (sem0, sem1)
    ocps = [None, None]
    for t in range(N_BATCH):
        if t + 1 < N_BATCH:
            icps[(t + 1) % 2] = pltpu.async_copy(
                cat_hbm.at[pl.ds(base + (t + 1) * BATCH, BATCH)],
                idxs[(t + 1) % 2], isems[(t + 1) % 2])
        if ocps[t % 2] is not None:
            ocps[t % 2].wait()
        icps[t % 2].wait()
        repack_batch(idxs[t % 2], accs[t % 2])
        ocps[t % 2] = pltpu.async_copy(
            accs[t % 2], out_hbm.at[pl.ds(base + t * BATCH, BATCH)],
            osems[t % 2])
    ocps[0].wait()
    ocps[1].wait()


@jax.jit
def _run(cat_tensor, tab_flat, wd):
    mesh = plsc.VectorSubcoreMesh(core_axis_name="c", subcore_axis_name="s")
    return pl.kernel(
        _body,
        out_type=jax.ShapeDtypeStruct((B, D_OUT), jnp.float32),
        mesh=mesh,
        scratch_types=[
            pltpu.VMEM((BATCH, 4), jnp.int32),
            pltpu.VMEM((BATCH, 4), jnp.int32),
            pltpu.VMEM((2, 1), jnp.float32),
            pltpu.VMEM((512 * 50,), jnp.float32),
            pltpu.VMEM((BATCH, D_OUT), jnp.float32),
            pltpu.VMEM((BATCH, D_OUT), jnp.float32),
            pltpu.SemaphoreType.DMA,
            pltpu.SemaphoreType.DMA,
            pltpu.SemaphoreType.DMA,
            pltpu.SemaphoreType.DMA,
        ],
        compiler_params=pltpu.CompilerParams(
            use_tc_tiling_on_sc=True, needs_layout_passes=False),
    )(cat_tensor, tab_flat, wd)


def kernel(cat_tensor, W_bus_id, W_station_id, W_time_period, W_direction):
    tab_flat = jnp.concatenate(
        [W_bus_id.reshape(-1), W_station_id.reshape(-1),
         W_time_period.reshape(-1)])
    return _run(cat_tensor, tab_flat, W_direction)


# speedup vs baseline: 1.6523x; 1.0291x over previous
import functools

import jax
import jax.numpy as jnp
from jax import lax
from jax.experimental import pallas as pl
from jax.experimental.pallas import tpu as pltpu
from jax.experimental.pallas import tpu_sc as plsc

B = 16384
D_OUT = 151
NC, NS, NL = 2, 16, 16
NW = NC * NS
B_PER_W = B // NW
BATCH = 128
N_BATCH = B_PER_W // BATCH
ROW_OFF = (0, 128, 384)


def _body(cat_hbm, tab_hbm, wd_hbm, out_hbm,
          idxa_v, idxb_v, dir_v, tab_v, acc0_v, acc1_v,
          sema, semb, sem0, sem1):
    wid = lax.axis_index("s") * NC + lax.axis_index("c")
    base = wid * B_PER_W

    idxs = (idxa_v, idxb_v)
    isems = (sema, semb)
    icps = [None, None]
    icps[0] = pltpu.async_copy(cat_hbm.at[pl.ds(base, BATCH)], idxa_v, sema)
    pltpu.sync_copy(wd_hbm, dir_v)
    pltpu.sync_copy(tab_hbm, tab_v)

    iota = lax.iota(jnp.int32, NL)
    zero = jnp.zeros((NL,), jnp.int32)
    kcol = [zero, zero + 1, zero + 2, zero + 3]
    cols = [iota, iota + 16, iota + 32]
    dcols = [[iota + (k * 50 + c * 16) for c in range(3)] for k in range(3)]

    def repack_batch(idx_v, acc_v):
        @plsc.parallel_loop(0, BATCH, unroll=2)
        def _rows(b):
            bv = jnp.full((NL,), b, jnp.int32)
            for k in range(3):
                tk = plsc.load_gather(idx_v, [bv, kcol[k]])
                tq = tk * 50 + (ROW_OFF[k] * 50)
                for c in range(3):
                    v = plsc.load_gather(tab_v, [tq + cols[c]])
                    plsc.store_scatter(acc_v, [bv, dcols[k][c]], v)

        @plsc.parallel_loop(0, BATCH // NL, unroll=2)
        def _tails(j):
            rl = j * NL + iota
            for k in range(3):
                tkv = plsc.load_gather(idx_v, [rl, kcol[k]])
                tkq = tkv * 50 + (ROW_OFF[k] * 50)
                for c in (48, 49):
                    v = plsc.load_gather(tab_v, [tkq + c])
                    plsc.store_scatter(
                        acc_v, [rl, jnp.full((NL,), k * 50 + c, jnp.int32)], v)
            dv = plsc.load_gather(idx_v, [rl, kcol[3]])
            dvals = plsc.load_gather(dir_v, [dv, zero])
            plsc.store_scatter(
                acc_v, [rl, jnp.full((NL,), 150, jnp.int32)], dvals)

    icps[1] = pltpu.async_copy(
        cat_hbm.at[pl.ds(base + BATCH, BATCH)], idxb_v, semb)

    def wait_idx(buf, sem):
        pltpu.make_async_copy(
            cat_hbm.at[pl.ds(0, BATCH)], buf, sem).wait()

    def wait_out(buf, sem):
        pltpu.make_async_copy(
            buf, out_hbm.at[pl.ds(0, BATCH)], sem).wait()

    def pair(u, _):
        off0 = base + u * (2 * BATCH)
        for half, (ibuf, isem, abuf, osem) in enumerate(
                ((idxa_v, sema, acc0_v, sem0), (idxb_v, semb, acc1_v, sem1))):
            wait_idx(ibuf, isem)

            @pl.when(u > 0)
            def _():
                wait_out(abuf, osem)

            repack_batch(ibuf, abuf)
            pref = jnp.minimum(off0 + (2 + half) * BATCH, B - BATCH)
            pltpu.async_copy(cat_hbm.at[pl.ds(pref, BATCH)], ibuf, isem)
            pltpu.async_copy(
                abuf, out_hbm.at[pl.ds(off0 + half * BATCH, BATCH)], osem)
        return 0

    lax.fori_loop(0, N_BATCH // 2, pair, 0)
    wait_out(acc0_v, sem0)
    wait_out(acc1_v, sem1)
    wait_idx(idxa_v, sema)
    wait_idx(idxb_v, semb)


@jax.jit
def _run(cat_tensor, tab_flat, wd):
    mesh = plsc.VectorSubcoreMesh(core_axis_name="c", subcore_axis_name="s")
    return pl.kernel(
        _body,
        out_type=jax.ShapeDtypeStruct((B, D_OUT), jnp.float32),
        mesh=mesh,
        scratch_types=[
            pltpu.VMEM((BATCH, 4), jnp.int32),
            pltpu.VMEM((BATCH, 4), jnp.int32),
            pltpu.VMEM((2, 1), jnp.float32),
            pltpu.VMEM((512 * 50,), jnp.float32),
            pltpu.VMEM((BATCH, D_OUT), jnp.float32),
            pltpu.VMEM((BATCH, D_OUT), jnp.float32),
            pltpu.SemaphoreType.DMA,
            pltpu.SemaphoreType.DMA,
            pltpu.SemaphoreType.DMA,
            pltpu.SemaphoreType.DMA,
        ],
        compiler_params=pltpu.CompilerParams(
            use_tc_tiling_on_sc=True, needs_layout_passes=False),
    )(cat_tensor, tab_flat, wd)


def kernel(cat_tensor, W_bus_id, W_station_id, W_time_period, W_direction):
    tab_flat = jnp.concatenate(
        [W_bus_id.reshape(-1), W_station_id.reshape(-1),
         W_time_period.reshape(-1)])
    return _run(cat_tensor, tab_flat, W_direction)
